# baseline (speedup 1.0000x reference)
import jax
import jax.numpy as jnp
from jax import lax
from jax.experimental import pallas as pl
from jax.experimental.pallas import tpu as pltpu
from jax.experimental.pallas import tpu_sc as plsc

NS = 10000
E = 320000
H = 8
HD = 16
D = 128
FEAT = 50
FFNDIM = 512

NCORE = 2
NSUB = 16
NWK = NCORE * NSUB
EPW = E // NWK
CH = 80
NCH = EPW // CH
NSP = 10240
RPT = NSP // NSUB
EFROWS = E * H // D
EFWPC = CH * H
CH2 = 40
NCH2 = EPW // CH2


def _dense_pre_body(wn_ref, s_ref, wfc_ref, u_ref, v_ref,
                    zw_ref, aw_ref, as_ref, maw_ref, mas_ref):
    wn = wn_ref[...]
    sv = s_ref[...]
    zw_ref[...] = jnp.dot(wn, wfc_ref[...], preferred_element_type=jnp.float32)
    aw = jnp.dot(wn, u_ref[...], preferred_element_type=jnp.float32)
    asv = jnp.dot(sv, v_ref[...], preferred_element_type=jnp.float32)
    aw_ref[...] = aw
    as_ref[...] = asv
    maw_ref[...] = jnp.max(aw, axis=0, keepdims=True)
    mas_ref[...] = jnp.max(asv, axis=0, keepdims=True)


def _ef_body(vf_ref, e_ref, ef_ref, cm_ref):
    i = pl.program_id(0)
    z = jnp.dot(vf_ref[...], e_ref[...], preferred_element_type=jnp.float32)
    ef_ref[...] = z
    m = jnp.broadcast_to(jnp.max(z, axis=1, keepdims=True), (H, D))

    @pl.when(i == 0)
    def _():
        cm_ref[...] = m

    @pl.when(i != 0)
    def _():
        cm_ref[...] = jnp.maximum(cm_ref[...], m)


def _ring_schedule(nch, issue_g, wait_g, compute, wait_s, nsets=3):
    p = nsets - 1
    for i in range(p):
        issue_g(i, i)

    def body(j, carry):
        for k in range(nsets):
            c = nsets * j + k
            sg = (k + p) % nsets
            wait_g(c, k)
            compute(c, k)
            if k == 0:
                @pl.when(j > 0)
                def _():
                    wait_s(c - 1, sg)
            else:
                wait_s(c - 1, sg)
            issue_g(c + p, sg)
        return carry

    nfull = (nch - p) // nsets
    t0 = nsets * nfull
    lax.fori_loop(0, nfull, body, 0)
    for c in range(t0, nch):
        k = c % nsets
        wait_g(c, k)
        compute(c, k)
        wait_s(c - 1, (c - 1) % nsets)
        if c + p < nch:
            issue_g(c + p, (c + p) % nsets)
    wait_s(nch - 1, (nch - 1) % nsets)


def _sc_pass1_body(aw_hbm, as_hbm, ef_hbm, eidx_hbm, cpat_hbm, z8_hbm,
                   dpart_hbm, ex_hbm,
                   denom, src_idx, dst_idx, aw_b, as_b, ef_b, ex_b,
                   cpat_v, semg, sems, semw):
    cid = lax.axis_index("c")
    sid = lax.axis_index("s")
    wid = cid * NSUB + sid
    pltpu.sync_copy(eidx_hbm.at[0, wid], src_idx)
    pltpu.sync_copy(eidx_hbm.at[1, wid], dst_idx)
    pltpu.sync_copy(cpat_hbm, cpat_v)
    pltpu.sync_copy(z8_hbm.at[pl.ds(sid * RPT, RPT)],
                    denom.at[pl.ds(sid * RPT, RPT)])
    plsc.subcore_barrier()
    lanes = lax.iota(jnp.int32, 16)
    rpat = lanes // 8
    cpat8 = lanes % 8
    cshift = cpat_v[...]

    def issue_g(c, S):
        pltpu.async_copy(aw_hbm.at[src_idx.at[c]], aw_b[S], semg[S])
        pltpu.async_copy(as_hbm.at[dst_idx.at[c]], as_b[S], semg[S])
        g = wid * EPW + c * CH
        pltpu.async_copy(ef_hbm.at[:, pl.ds(g, CH)], ef_b[S], semg[S])

    def wait_g(c, S):
        pltpu.make_async_copy(aw_hbm.at[src_idx.at[c]], aw_b[S],
                              semg[S]).wait()
        pltpu.make_async_copy(as_hbm.at[dst_idx.at[c]], as_b[S],
                              semg[S]).wait()
        g = wid * EPW + c * CH
        pltpu.make_async_copy(ef_hbm.at[:, pl.ds(g, CH)], ef_b[S],
                              semg[S]).wait()

    def compute(c, S):
        for h in range(H):
            ch = cshift[h]
            hsp = jnp.full((16,), h, jnp.int32)
            for i in range(CH // 16):
                eidx16 = lanes + 16 * i
                awv = plsc.load_gather(aw_b[S], [eidx16, hsp])
                asv = plsc.load_gather(as_b[S], [eidx16, hsp])
                efv = ef_b[S][h, pl.ds(i * 16, 16)]
                x = awv + asv + efv
                ex = jnp.exp(jnp.maximum(x, 0.01 * x) - ch)
                plsc.store_scatter(ex_b[S], [eidx16, hsp], ex)
        pltpu.async_copy(ex_b[S], denom.at[dst_idx.at[c]], sems[S], add=True)
        pltpu.async_copy(ex_b[S], ex_hbm.at[wid, pl.ds(c * CH, CH)], semw[S])

    def wait_s(c, S):
        pltpu.make_async_copy(ex_b[S], denom.at[dst_idx.at[c]],
                              sems[S]).wait()
        pltpu.make_async_copy(ex_b[S], ex_hbm.at[wid, pl.ds(c * CH, CH)],
                              semw[S]).wait()

    _ring_schedule(NCH, issue_g, wait_g, compute, wait_s)
    plsc.subcore_barrier()
    pltpu.sync_copy(denom.at[pl.ds(sid * RPT, RPT)],
                    dpart_hbm.at[cid, pl.ds(sid * RPT, RPT)])


def _sc_pass2_body(zw_hbm, ex_hbm, eidx_hbm, z128_hbm,
                   nb_hbm,
                   nbacc, src_idx, dst_idx, zw_b, ex_b,
                   semg, sems):
    cid = lax.axis_index("c")
    sid = lax.axis_index("s")
    wid = cid * NSUB + sid
    pltpu.sync_copy(eidx_hbm.at[0, wid], src_idx)
    pltpu.sync_copy(eidx_hbm.at[1, wid], dst_idx)
    pltpu.sync_copy(z128_hbm.at[pl.ds(sid * RPT, RPT)],
                    nbacc.at[pl.ds(sid * RPT, RPT)])
    lanes = lax.iota(jnp.int32, 16)
    rpat = lanes // 8
    cpat8 = lanes % 8
    plsc.subcore_barrier()

    def issue_g(c, S):
        pltpu.async_copy(zw_hbm.at[src_idx.at[c]], zw_b[S], semg[S])
        pltpu.async_copy(ex_hbm.at[wid, pl.ds(c * CH2, CH2)],
                         ex_b[S].at[pl.ds(0, CH2)], semg[S])

    def wait_g(c, S):
        pltpu.make_async_copy(zw_hbm.at[src_idx.at[c]], zw_b[S],
                              semg[S]).wait()
        pltpu.make_async_copy(ex_hbm.at[wid, pl.ds(c * CH2, CH2)],
                              ex_b[S].at[pl.ds(0, CH2)], semg[S]).wait()

    def compute(c, S):
        def erow(e2, cc):
            av = plsc.load_gather(ex_b[S], [rpat + e2, cpat8])
            for h in range(H):
                a = av[h]
                v = zw_b[S][e2, pl.ds(h * HD, HD)]
                zw_b[S][e2, pl.ds(h * HD, HD)] = v * a
            return cc

        lax.fori_loop(0, CH2, erow, 0, unroll=4)
        pltpu.async_copy(zw_b[S], nbacc.at[dst_idx.at[c]], sems[S], add=True)

    def wait_s(c, S):
        pltpu.make_async_copy(zw_b[S], nbacc.at[dst_idx.at[c]],
                              sems[S]).wait()

    _ring_schedule(NCH2, issue_g, wait_g, compute, wait_s)
    plsc.subcore_barrier()
    pltpu.sync_copy(nbacc.at[pl.ds(sid * RPT, RPT)],
                    nb_hbm.at[cid, pl.ds(sid * RPT, RPT)])


def _build_pass1():
    f32 = jnp.float32
    mesh = plsc.VectorSubcoreMesh(core_axis_name="c", subcore_axis_name="s")
    return pl.kernel(
        _sc_pass1_body,
        out_type=(
            jax.ShapeDtypeStruct((NCORE, NSP, H), f32),
            jax.ShapeDtypeStruct((NWK, EPW, H), f32),
        ),
        mesh=mesh,
        compiler_params=pltpu.CompilerParams(use_tc_tiling_on_sc=False,
                                             needs_layout_passes=False),
        scratch_types=[
            pltpu.VMEM_SHARED((NSP, H), f32),
            pltpu.VMEM((NCH, CH), jnp.int32),
            pltpu.VMEM((NCH, CH), jnp.int32),
            [pltpu.VMEM((CH, H), f32) for _ in range(3)],
            [pltpu.VMEM((CH, H), f32) for _ in range(3)],
            [pltpu.VMEM((H, CH), f32) for _ in range(3)],
            [pltpu.VMEM((CH, H), f32) for _ in range(3)],
            pltpu.VMEM((16,), f32),
            [pltpu.SemaphoreType.DMA for _ in range(3)],
            [pltpu.SemaphoreType.DMA for _ in range(3)],
            [pltpu.SemaphoreType.DMA for _ in range(3)],
        ],
    )


def _build_pass2():
    f32 = jnp.float32
    mesh = plsc.VectorSubcoreMesh(core_axis_name="c", subcore_axis_name="s")
    return pl.kernel(
        _sc_pass2_body,
        out_type=jax.ShapeDtypeStruct((NCORE, NSP, D), f32),
        mesh=mesh,
        compiler_params=pltpu.CompilerParams(use_tc_tiling_on_sc=False,
                                             needs_layout_passes=False),
        scratch_types=[
            pltpu.VMEM_SHARED((NSP, D), f32),
            pltpu.VMEM((NCH2, CH2), jnp.int32),
            pltpu.VMEM((NCH2, CH2), jnp.int32),
            [pltpu.VMEM((CH2, D), f32) for _ in range(3)],
            [pltpu.VMEM((CH2 + 2, H), f32) for _ in range(3)],
            [pltpu.SemaphoreType.DMA for _ in range(3)],
            [pltpu.SemaphoreType.DMA for _ in range(3)],
        ],
    )


def _final_body(s_ref, n0_ref, n1_ref, d_ref, rep_ref, w1_ref, b1_ref,
                w2_ref, b2_ref, g_ref, be_ref, o_ref):
    rd = jnp.dot(1.0 / (d_ref[0] + d_ref[1] + 1e-9), rep_ref[...],
                 preferred_element_type=jnp.float32)
    x = s_ref[...] + (n0_ref[...] + n1_ref[...]) * rd
    hv = jnp.where(x > 0, x, jnp.exp(x) - 1.0)
    mu = jnp.mean(hv, axis=1, keepdims=True)
    t = hv - mu
    var = jnp.mean(t * t, axis=1, keepdims=True)
    ln = t / jnp.sqrt(var + 1e-6) * g_ref[...] + be_ref[...]
    f1 = jnp.maximum(
        jnp.dot(ln, w1_ref[...], preferred_element_type=jnp.float32)
        + b1_ref[...], 0.0)
    f2 = jnp.dot(f1, w2_ref[...], preferred_element_type=jnp.float32) \
        + b2_ref[...]
    o_ref[...] = hv + f2


def kernel(w, s, edge_feat, edge_index, Wfc, Wfeat, attn, W1, b1, W2, b2,
           gamma, beta):
    f32 = jnp.float32
    wn = w[:NS]
    Wfc_cat = jnp.transpose(Wfc, (1, 0, 2)).reshape(D, D)
    U = jnp.einsum('hdk,hk->dh', Wfc, attn[:, :HD])
    V = jnp.einsum('hdk,hk->dh', Wfc, attn[:, 2 * HD:])
    VfT = jnp.einsum('hfk,hk->hf', Wfeat, attn[:, HD:2 * HD])
    eT = edge_feat.T
    eidx = edge_index.reshape(2, NWK, NCH, CH)
    z8 = jnp.zeros((NSP, H), f32)
    z128 = jnp.zeros((NSP, D), f32)

    ZW, AW, AS, mAW, mAS = pl.pallas_call(
        _dense_pre_body,
        out_shape=(
            jax.ShapeDtypeStruct((NS, D), f32),
            jax.ShapeDtypeStruct((NS, H), f32),
            jax.ShapeDtypeStruct((NS, H), f32),
            jax.ShapeDtypeStruct((1, H), f32),
            jax.ShapeDtypeStruct((1, H), f32),
        ),
    )(wn, s, Wfc_cat, U, V)

    NEB = 20
    BE = E // NEB
    EFT, cmax = pl.pallas_call(
        _ef_body,
        grid=(NEB,),
        in_specs=[
            pl.BlockSpec((H, FEAT), lambda i: (0, 0)),
            pl.BlockSpec((FEAT, BE), lambda i: (0, i)),
        ],
        out_specs=(
            pl.BlockSpec((H, BE), lambda i: (0, i)),
            pl.BlockSpec((H, D), lambda i: (0, 0)),
        ),
        out_shape=(
            jax.ShapeDtypeStruct((H, E), f32),
            jax.ShapeDtypeStruct((H, D), f32),
        ),
    )(VfT, eT)

    c8 = jnp.maximum(mAW[0] + mAS[0] + cmax[:, 0], 0.0)
    cpat = jnp.tile(c8, 2)

    dpart, exall = _build_pass1()(AW, AS, EFT, eidx, cpat, z8)

    rep = jnp.kron(jnp.eye(H, dtype=f32), jnp.ones((1, HD), f32))
    eidx2 = edge_index.reshape(2, NWK, NCH2, CH2)
    nb = _build_pass2()(ZW, exall, eidx2, z128)
    NRB = 10
    RB = NS // NRB
    out = pl.pallas_call(
        _final_body,
        grid=(NRB,),
        in_specs=[
            pl.BlockSpec((RB, D), lambda i: (i, 0)),
            pl.BlockSpec((RB, D), lambda i: (i, 0)),
            pl.BlockSpec((RB, D), lambda i: (i, 0)),
            pl.BlockSpec((2, RB, H), lambda i: (0, i, 0)),
            pl.BlockSpec((H, D), lambda i: (0, 0)),
            pl.BlockSpec((D, FFNDIM), lambda i: (0, 0)),
            pl.BlockSpec((1, FFNDIM), lambda i: (0, 0)),
            pl.BlockSpec((FFNDIM, D), lambda i: (0, 0)),
            pl.BlockSpec((1, D), lambda i: (0, 0)),
            pl.BlockSpec((1, D), lambda i: (0, 0)),
            pl.BlockSpec((1, D), lambda i: (0, 0)),
        ],
        out_specs=pl.BlockSpec((RB, D), lambda i: (i, 0)),
        out_shape=jax.ShapeDtypeStruct((NS, D), f32),
    )(s, nb[0, :NS], nb[1, :NS], dpart, rep, W1, b1.reshape(1, FFNDIM), W2,
      b2.reshape(1, D), gamma.reshape(1, D), beta.reshape(1, D))
    return out


# speedup vs baseline: 1.1195x; 1.1195x over previous
import jax
import jax.numpy as jnp
from jax import lax
from jax.experimental import pallas as pl
from jax.experimental.pallas import tpu as pltpu
from jax.experimental.pallas import tpu_sc as plsc

NS = 10000
E = 320000
H = 8
HD = 16
D = 128
FEAT = 50
FFNDIM = 512

NCORE = 2
NSUB = 16
NWK = NCORE * NSUB
EPW = E // NWK
CH = 80
NCH = EPW // CH
NSP = 10240
RPT = NSP // NSUB
EFROWS = E * H // D
EFWPC = CH * H
CH2 = 40
NCH2 = EPW // CH2


def _dense_pre_body(wn_ref, s_ref, wfc_ref, u_ref, v_ref,
                    zw_ref, aw_ref, as_ref, maw_ref, mas_ref):
    wn = wn_ref[...]
    sv = s_ref[...]
    zw_ref[...] = jnp.dot(wn, wfc_ref[...], preferred_element_type=jnp.float32)
    aw = jnp.dot(wn, u_ref[...], preferred_element_type=jnp.float32)
    asv = jnp.dot(sv, v_ref[...], preferred_element_type=jnp.float32)
    aw_ref[...] = aw
    as_ref[...] = asv
    maw_ref[...] = jnp.max(aw, axis=0, keepdims=True)
    mas_ref[...] = jnp.max(asv, axis=0, keepdims=True)


def _ef_body(vf_ref, e_ref, ef_ref, cm_ref):
    i = pl.program_id(0)
    z = jnp.dot(vf_ref[...], e_ref[...], preferred_element_type=jnp.float32)
    ef_ref[...] = z
    m = jnp.broadcast_to(jnp.max(z, axis=1, keepdims=True), (H, D))

    @pl.when(i == 0)
    def _():
        cm_ref[...] = m

    @pl.when(i != 0)
    def _():
        cm_ref[...] = jnp.maximum(cm_ref[...], m)


def _ring_schedule(nch, issue_g, wait_g, compute, wait_s, nsets=3):
    p = nsets - 1
    for i in range(p):
        issue_g(i, i)

    def body(j, carry):
        for k in range(nsets):
            c = nsets * j + k
            sg = (k + p) % nsets
            wait_g(c, k)
            compute(c, k)
            if k == 0:
                @pl.when(j > 0)
                def _():
                    wait_s(c - 1, sg)
            else:
                wait_s(c - 1, sg)
            issue_g(c + p, sg)
        return carry

    nfull = (nch - p) // nsets
    t0 = nsets * nfull
    lax.fori_loop(0, nfull, body, 0)
    for c in range(t0, nch):
        k = c % nsets
        wait_g(c, k)
        compute(c, k)
        wait_s(c - 1, (c - 1) % nsets)
        if c + p < nch:
            issue_g(c + p, (c + p) % nsets)
    wait_s(nch - 1, (nch - 1) % nsets)


def _sc_pass1_body(aw_hbm, as_hbm, ef_hbm, eidx_hbm, cpat_hbm, z8_hbm,
                   dpart_hbm, ex_hbm,
                   denom, src_idx, dst_idx, aw_b, as_b, ef_b, ex_b,
                   cpat_v, semg, sems, semw):
    cid = lax.axis_index("c")
    sid = lax.axis_index("s")
    wid = cid * NSUB + sid
    pltpu.sync_copy(eidx_hbm.at[0, wid], src_idx)
    pltpu.sync_copy(eidx_hbm.at[1, wid], dst_idx)
    pltpu.sync_copy(cpat_hbm, cpat_v)
    pltpu.sync_copy(z8_hbm.at[pl.ds(sid * RPT, RPT)],
                    denom.at[pl.ds(sid * RPT, RPT)])
    plsc.subcore_barrier()
    lanes = lax.iota(jnp.int32, 16)
    rpat = lanes // 8
    cpat8 = lanes % 8
    cshift = cpat_v[...]

    def issue_g(c, S):
        pltpu.async_copy(aw_hbm.at[src_idx.at[c]], aw_b[S], semg[S])
        pltpu.async_copy(as_hbm.at[dst_idx.at[c]], as_b[S], semg[S])
        g = wid * EPW + c * CH
        pltpu.async_copy(ef_hbm.at[:, pl.ds(g, CH)], ef_b[S], semg[S])

    def wait_g(c, S):
        pltpu.make_async_copy(aw_hbm.at[src_idx.at[c]], aw_b[S],
                              semg[S]).wait()
        pltpu.make_async_copy(as_hbm.at[dst_idx.at[c]], as_b[S],
                              semg[S]).wait()
        g = wid * EPW + c * CH
        pltpu.make_async_copy(ef_hbm.at[:, pl.ds(g, CH)], ef_b[S],
                              semg[S]).wait()

    def compute(c, S):
        for h in range(H):
            ch = cshift[h]
            hsp = jnp.full((16,), h, jnp.int32)
            for i in range(CH // 16):
                eidx16 = lanes + 16 * i
                awv = plsc.load_gather(aw_b[S], [eidx16, hsp])
                asv = plsc.load_gather(as_b[S], [eidx16, hsp])
                efv = ef_b[S][h, pl.ds(i * 16, 16)]
                x = awv + asv + efv
                ex = jnp.exp(jnp.maximum(x, 0.01 * x) - ch)
                plsc.store_scatter(ex_b[S], [eidx16, hsp], ex)
        pltpu.async_copy(ex_b[S], denom.at[dst_idx.at[c]], sems[S], add=True)
        pltpu.async_copy(ex_b[S], ex_hbm.at[wid, pl.ds(c * CH, CH)], semw[S])

    def wait_s(c, S):
        pltpu.make_async_copy(ex_b[S], denom.at[dst_idx.at[c]],
                              sems[S]).wait()
        pltpu.make_async_copy(ex_b[S], ex_hbm.at[wid, pl.ds(c * CH, CH)],
                              semw[S]).wait()

    _ring_schedule(NCH, issue_g, wait_g, compute, wait_s)
    plsc.subcore_barrier()
    pltpu.sync_copy(denom.at[pl.ds(sid * RPT, RPT)],
                    dpart_hbm.at[cid, pl.ds(sid * RPT, RPT)])


def _sc_pass2_body(zw_hbm, ex_hbm, eidx_hbm, z128_hbm,
                   nb_hbm,
                   nbacc, src_idx, dst_idx, zw_b, ex_b,
                   semg, sems):
    cid = lax.axis_index("c")
    sid = lax.axis_index("s")
    wid = cid * NSUB + sid
    pltpu.sync_copy(eidx_hbm.at[0, wid], src_idx)
    pltpu.sync_copy(eidx_hbm.at[1, wid], dst_idx)
    pltpu.sync_copy(z128_hbm.at[pl.ds(sid * RPT, RPT)],
                    nbacc.at[pl.ds(sid * RPT, RPT)])
    lanes = lax.iota(jnp.int32, 16)
    rpat = lanes // 8
    cpat8 = lanes % 8
    plsc.subcore_barrier()

    def issue_g(c, S):
        pltpu.async_copy(zw_hbm.at[src_idx.at[c]], zw_b[S], semg[S])
        pltpu.async_copy(ex_hbm.at[wid, pl.ds(c * CH2, CH2)],
                         ex_b[S].at[pl.ds(0, CH2)], semg[S])

    def wait_g(c, S):
        pltpu.make_async_copy(zw_hbm.at[src_idx.at[c]], zw_b[S],
                              semg[S]).wait()
        pltpu.make_async_copy(ex_hbm.at[wid, pl.ds(c * CH2, CH2)],
                              ex_b[S].at[pl.ds(0, CH2)], semg[S]).wait()

    def compute(c, S):
        def erow(e2, cc):
            av = plsc.load_gather(ex_b[S], [rpat + e2, cpat8])
            for h in range(H):
                a = av[h]
                v = zw_b[S][e2, pl.ds(h * HD, HD)]
                zw_b[S][e2, pl.ds(h * HD, HD)] = v * a
            return cc

        lax.fori_loop(0, CH2, erow, 0, unroll=4)
        pltpu.async_copy(zw_b[S], nbacc.at[dst_idx.at[c]], sems[S], add=True)

    def wait_s(c, S):
        pltpu.make_async_copy(zw_b[S], nbacc.at[dst_idx.at[c]],
                              sems[S]).wait()

    _ring_schedule(NCH2, issue_g, wait_g, compute, wait_s, nsets=4)
    plsc.subcore_barrier()
    pltpu.sync_copy(nbacc.at[pl.ds(sid * RPT, RPT)],
                    nb_hbm.at[cid, pl.ds(sid * RPT, RPT)])


def _build_pass1():
    f32 = jnp.float32
    mesh = plsc.VectorSubcoreMesh(core_axis_name="c", subcore_axis_name="s")
    return pl.kernel(
        _sc_pass1_body,
        out_type=(
            jax.ShapeDtypeStruct((NCORE, NSP, H), f32),
            jax.ShapeDtypeStruct((NWK, EPW, H), f32),
        ),
        mesh=mesh,
        compiler_params=pltpu.CompilerParams(use_tc_tiling_on_sc=False,
                                             needs_layout_passes=False),
        scratch_types=[
            pltpu.VMEM_SHARED((NSP, H), f32),
            pltpu.VMEM((NCH, CH), jnp.int32),
            pltpu.VMEM((NCH, CH), jnp.int32),
            [pltpu.VMEM((CH, H), f32) for _ in range(3)],
            [pltpu.VMEM((CH, H), f32) for _ in range(3)],
            [pltpu.VMEM((H, CH), f32) for _ in range(3)],
            [pltpu.VMEM((CH, H), f32) for _ in range(3)],
            pltpu.VMEM((16,), f32),
            [pltpu.SemaphoreType.DMA for _ in range(3)],
            [pltpu.SemaphoreType.DMA for _ in range(3)],
            [pltpu.SemaphoreType.DMA for _ in range(3)],
        ],
    )


def _build_pass2():
    f32 = jnp.float32
    mesh = plsc.VectorSubcoreMesh(core_axis_name="c", subcore_axis_name="s")
    return pl.kernel(
        _sc_pass2_body,
        out_type=jax.ShapeDtypeStruct((NCORE, NSP, D), f32),
        mesh=mesh,
        compiler_params=pltpu.CompilerParams(use_tc_tiling_on_sc=False,
                                             needs_layout_passes=False),
        scratch_types=[
            pltpu.VMEM_SHARED((NSP, D), f32),
            pltpu.VMEM((NCH2, CH2), jnp.int32),
            pltpu.VMEM((NCH2, CH2), jnp.int32),
            [pltpu.VMEM((CH2, D), f32) for _ in range(4)],
            [pltpu.VMEM((CH2 + 2, H), f32) for _ in range(4)],
            [pltpu.SemaphoreType.DMA for _ in range(4)],
            [pltpu.SemaphoreType.DMA for _ in range(4)],
        ],
    )


def _final_body(s_ref, n0_ref, n1_ref, d_ref, rep_ref, w1_ref, b1_ref,
                w2_ref, b2_ref, g_ref, be_ref, o_ref):
    rd = jnp.dot(1.0 / (d_ref[0] + d_ref[1] + 1e-9), rep_ref[...],
                 preferred_element_type=jnp.float32)
    x = s_ref[...] + (n0_ref[...] + n1_ref[...]) * rd
    hv = jnp.where(x > 0, x, jnp.exp(x) - 1.0)
    mu = jnp.mean(hv, axis=1, keepdims=True)
    t = hv - mu
    var = jnp.mean(t * t, axis=1, keepdims=True)
    ln = t / jnp.sqrt(var + 1e-6) * g_ref[...] + be_ref[...]
    f1 = jnp.maximum(
        jnp.dot(ln, w1_ref[...], preferred_element_type=jnp.float32)
        + b1_ref[...], 0.0)
    f2 = jnp.dot(f1, w2_ref[...], preferred_element_type=jnp.float32) \
        + b2_ref[...]
    o_ref[...] = hv + f2


def kernel(w, s, edge_feat, edge_index, Wfc, Wfeat, attn, W1, b1, W2, b2,
           gamma, beta):
    f32 = jnp.float32
    wn = w[:NS]
    Wfc_cat = jnp.transpose(Wfc, (1, 0, 2)).reshape(D, D)
    U = jnp.einsum('hdk,hk->dh', Wfc, attn[:, :HD])
    V = jnp.einsum('hdk,hk->dh', Wfc, attn[:, 2 * HD:])
    VfT = jnp.einsum('hfk,hk->hf', Wfeat, attn[:, HD:2 * HD])
    eT = edge_feat.T
    eidx = edge_index.reshape(2, NWK, NCH, CH)
    z8 = jnp.zeros((NSP, H), f32)
    z128 = jnp.zeros((NSP, D), f32)

    ZW, AW, AS, mAW, mAS = pl.pallas_call(
        _dense_pre_body,
        out_shape=(
            jax.ShapeDtypeStruct((NS, D), f32),
            jax.ShapeDtypeStruct((NS, H), f32),
            jax.ShapeDtypeStruct((NS, H), f32),
            jax.ShapeDtypeStruct((1, H), f32),
            jax.ShapeDtypeStruct((1, H), f32),
        ),
    )(wn, s, Wfc_cat, U, V)

    NEB = 20
    BE = E // NEB
    EFT, cmax = pl.pallas_call(
        _ef_body,
        grid=(NEB,),
        in_specs=[
            pl.BlockSpec((H, FEAT), lambda i: (0, 0)),
            pl.BlockSpec((FEAT, BE), lambda i: (0, i)),
        ],
        out_specs=(
            pl.BlockSpec((H, BE), lambda i: (0, i)),
            pl.BlockSpec((H, D), lambda i: (0, 0)),
        ),
        out_shape=(
            jax.ShapeDtypeStruct((H, E), f32),
            jax.ShapeDtypeStruct((H, D), f32),
        ),
    )(VfT, eT)

    c8 = jnp.maximum(mAW[0] + mAS[0] + cmax[:, 0], 0.0)
    cpat = jnp.tile(c8, 2)

    dpart, exall = _build_pass1()(AW, AS, EFT, eidx, cpat, z8)

    rep = jnp.kron(jnp.eye(H, dtype=f32), jnp.ones((1, HD), f32))
    eidx2 = edge_index.reshape(2, NWK, NCH2, CH2)
    nb = _build_pass2()(ZW, exall, eidx2, z128)
    NRB = 10
    RB = NS // NRB
    out = pl.pallas_call(
        _final_body,
        grid=(NRB,),
        in_specs=[
            pl.BlockSpec((RB, D), lambda i: (i, 0)),
            pl.BlockSpec((RB, D), lambda i: (i, 0)),
            pl.BlockSpec((RB, D), lambda i: (i, 0)),
            pl.BlockSpec((2, RB, H), lambda i: (0, i, 0)),
            pl.BlockSpec((H, D), lambda i: (0, 0)),
            pl.BlockSpec((D, FFNDIM), lambda i: (0, 0)),
            pl.BlockSpec((1, FFNDIM), lambda i: (0, 0)),
            pl.BlockSpec((FFNDIM, D), lambda i: (0, 0)),
            pl.BlockSpec((1, D), lambda i: (0, 0)),
            pl.BlockSpec((1, D), lambda i: (0, 0)),
            pl.BlockSpec((1, D), lambda i: (0, 0)),
        ],
        out_specs=pl.BlockSpec((RB, D), lambda i: (i, 0)),
        out_shape=jax.ShapeDtypeStruct((NS, D), f32),
    )(s, nb[0, :NS], nb[1, :NS], dpart, rep, W1, b1.reshape(1, FFNDIM), W2,
      b2.reshape(1, D), gamma.reshape(1, D), beta.reshape(1, D))
    return out
